# baseline (speedup 1.0000x reference)
import functools

import jax
import jax.numpy as jnp
from jax import lax
from jax.experimental import pallas as pl
from jax.experimental.pallas import tpu as pltpu
from jax.experimental.pallas import tpu_sc as plsc

N = 10000
NP = 10240
E = 320000
D = 128
B = 256
NC = 2
NS = 16
CHUNK = 80
CCHUNK = 80
NBUF = 2
ROWS_PER_TILE = NP // NS

_DEG_CHUNKS = E // (NC * NS) // CHUNK
_CONV_CHUNKS = E // NS // CCHUNK
_NGRP = 25
_GRP_CHUNKS = _CONV_CHUNKS // _NGRP
assert _GRP_CHUNKS % NBUF == 0
_NQUAD = _CONV_CHUNKS // NBUF
_QPG = _GRP_CHUNKS // NBUF


@functools.cache
def _sc_kernels():
    mesh = plsc.VectorSubcoreMesh(core_axis_name="c", subcore_axis_name="s",
                                  num_cores=NC, num_subcores=NS)

    @functools.partial(
        pl.kernel,
        mesh=mesh,
        out_type=jax.ShapeDtypeStruct((NC, NP, D), jnp.float32),
        scratch_types=[
            pltpu.VMEM((_DEG_CHUNKS, CHUNK), jnp.int32),
            pltpu.VMEM((CHUNK, D), jnp.float32),
            pltpu.VMEM_SHARED((NP, D), jnp.float32),
        ],
    )
    def sc_degree(dst_hbm, zeros_hbm, ones_hbm, out_hbm, idx_v, ones_v, deg_sh):
        cid = lax.axis_index("c")
        sid = lax.axis_index("s")
        wid = cid * NS + sid
        row0 = sid * ROWS_PER_TILE
        pltpu.sync_copy(zeros_hbm, deg_sh.at[pl.ds(row0, ROWS_PER_TILE)])
        pltpu.sync_copy(ones_hbm, ones_v)
        pltpu.sync_copy(dst_hbm.at[wid], idx_v)
        plsc.subcore_barrier()

        def body(j, carry):
            pltpu.sync_copy(ones_v, deg_sh.at[idx_v.at[j]], add=True)
            return carry

        lax.fori_loop(0, _DEG_CHUNKS, body, 0)
        plsc.subcore_barrier()
        pltpu.sync_copy(deg_sh.at[pl.ds(row0, ROWS_PER_TILE)],
                        out_hbm.at[cid, pl.ds(row0, ROWS_PER_TILE)])

    @functools.partial(
        pl.kernel,
        mesh=mesh,
        out_type=jax.ShapeDtypeStruct((NC, NP, D), jnp.float32),
        scratch_types=[
            pltpu.VMEM((2, _GRP_CHUNKS, CCHUNK), jnp.int32),
            pltpu.VMEM((2, _GRP_CHUNKS, CCHUNK), jnp.int32),
            pltpu.VMEM((NBUF, CCHUNK, D), jnp.float32),
            [pltpu.SemaphoreType.DMA for _ in range(NBUF)],
            [pltpu.SemaphoreType.DMA for _ in range(NBUF)],
            pltpu.SemaphoreType.DMA,
            pltpu.VMEM_SHARED((NP, D), jnp.float32),
        ],
    )
    def sc_conv(hp_hbm, srcg_hbm, dstg_hbm, zeros_hbm, out_hbm,
                src_v, dst_v, rows, gsems, ssems, isem, acc_sh):
        cid = lax.axis_index("c")
        sid = lax.axis_index("s")
        row0 = sid * ROWS_PER_TILE
        pltpu.sync_copy(zeros_hbm, acc_sh.at[pl.ds(row0, ROWS_PER_TILE)])
        pltpu.sync_copy(srcg_hbm.at[cid, sid, 0], src_v.at[0])
        pltpu.sync_copy(dstg_hbm.at[sid, 0], dst_v.at[0])
        plsc.subcore_barrier()

        def quad(q, carry):
            ra = (q % _QPG) * NBUF
            grp = q // _QPG
            gpar = grp % 2
            npar = (grp + 1) % 2

            @pl.when((ra == 0) & (grp > 0))
            def _wait_idx():
                pltpu.make_async_copy(srcg_hbm.at[cid, sid, grp],
                                      src_v.at[gpar], isem).wait()
                pltpu.make_async_copy(dstg_hbm.at[sid, grp],
                                      dst_v.at[gpar], isem).wait()

            @pl.when((ra == 0) & (grp < _NGRP - 1))
            def _prefetch_idx():
                pltpu.async_copy(srcg_hbm.at[cid, sid, grp + 1],
                                 src_v.at[npar], isem)
                pltpu.async_copy(dstg_hbm.at[sid, grp + 1],
                                 dst_v.at[npar], isem)

            for k in range(NBUF):
                @pl.when(q > 0)
                def _drain_prev(k=k):
                    pltpu.make_async_copy(
                        rows.at[k], acc_sh.at[dst_v.at[gpar, ra + k]],
                        ssems[k]).wait()
                pltpu.async_copy(hp_hbm.at[src_v.at[gpar, ra + k]],
                                 rows.at[k], gsems[k])
            for k in range(NBUF):
                pltpu.make_async_copy(hp_hbm.at[src_v.at[gpar, ra + k]],
                                      rows.at[k], gsems[k]).wait()
                pltpu.async_copy(rows.at[k], acc_sh.at[dst_v.at[gpar, ra + k]],
                                 ssems[k], add=True)
            return carry

        lax.fori_loop(0, _NQUAD, quad, 0)
        for k in range(NBUF):
            pltpu.make_async_copy(rows.at[k], acc_sh.at[dst_v.at[0, k]],
                                  ssems[k]).wait()
        plsc.subcore_barrier()
        pltpu.sync_copy(acc_sh.at[pl.ds(row0, ROWS_PER_TILE)],
                        out_hbm.at[cid, pl.ds(row0, ROWS_PER_TILE)])

    return sc_degree, sc_conv


_RB = 1024
_NRB = NP // _RB


def _tc_prep_body(deg_ref, x_ref, w_ref, dinv_ref, hp_ref):
    degf = deg_ref[0] + deg_ref[1]
    total = degf + 1.0
    dinv = lax.rsqrt(jnp.maximum(total, 1.0))
    dinv_ref[...] = dinv
    h = jnp.dot(x_ref[...], w_ref[0], preferred_element_type=jnp.float32,
                 precision=lax.Precision.HIGHEST)
    hp_ref[...] = (h * dinv)[None]


def _tc_prep(degp, x, w0):
    return pl.pallas_call(
        _tc_prep_body,
        grid=(NC, _NRB),
        in_specs=[
            pl.BlockSpec((NC, _RB, D), lambda c, i: (0, i, 0)),
            pl.BlockSpec((_RB, D), lambda c, i: (i, 0)),
            pl.BlockSpec((1, D, D), lambda c, i: (c, 0, 0)),
        ],
        out_specs=[
            pl.BlockSpec((_RB, D), lambda c, i: (i, 0)),
            pl.BlockSpec((1, _RB, D), lambda c, i: (c, i, 0)),
        ],
        out_shape=[
            jax.ShapeDtypeStruct((NP, D), jnp.float32),
            jax.ShapeDtypeStruct((NC, NP, D), jnp.float32),
        ],
    )(degp, x, w0)


def _combine(hp, seg, dinv, b):
    out = dinv * (hp + seg) + b
    nrm2 = jnp.sum(out * out, axis=-1, keepdims=True)
    return jnp.maximum(out * lax.rsqrt(jnp.maximum(nrm2, 1e-24)), 0.0)


def _tc_layer_body(hp_ref, seg_ref, dinv_ref, b_ref, w_ref, hpn_ref):
    xn = _combine(hp_ref[0], seg_ref[0], dinv_ref[...], b_ref[0])
    h = jnp.dot(xn, w_ref[0], preferred_element_type=jnp.float32,
                 precision=lax.Precision.HIGHEST)
    hpn_ref[...] = (h * dinv_ref[...])[None]


def _tc_layer(hp, seg, dinv, b, wn):
    return pl.pallas_call(
        _tc_layer_body,
        grid=(NC, _NRB),
        in_specs=[
            pl.BlockSpec((1, _RB, D), lambda c, i: (c, i, 0)),
            pl.BlockSpec((1, _RB, D), lambda c, i: (c, i, 0)),
            pl.BlockSpec((_RB, D), lambda c, i: (i, 0)),
            pl.BlockSpec((1, 1, D), lambda c, i: (c, 0, 0)),
            pl.BlockSpec((1, D, D), lambda c, i: (c, 0, 0)),
        ],
        out_specs=pl.BlockSpec((1, _RB, D), lambda c, i: (c, i, 0)),
        out_shape=jax.ShapeDtypeStruct((NC, NP, D), jnp.float32),
    )(hp, seg, dinv, b, wn)


def _tc_pool_body(hp_ref, seg_ref, dinv_ref, b_ref, batch_ref, pooled_ref):
    i = pl.program_id(1)
    xn = _combine(hp_ref[0], seg_ref[0], dinv_ref[...], b_ref[0])
    bvec = batch_ref[0, 0]
    gids = lax.broadcasted_iota(jnp.int32, (B, _RB), 0)
    oh_t = jnp.where(gids == bvec[None, :], 1.0, 0.0)
    contrib = jnp.dot(oh_t, xn, preferred_element_type=jnp.float32,
                 precision=lax.Precision.HIGHEST)

    @pl.when(i == 0)
    def _init():
        pooled_ref[...] = contrib[None]

    @pl.when(i != 0)
    def _acc():
        pooled_ref[...] = pooled_ref[...] + contrib[None]


def _tc_pool(hp, seg, dinv, b, batch3):
    return pl.pallas_call(
        _tc_pool_body,
        grid=(NC, _NRB),
        in_specs=[
            pl.BlockSpec((1, _RB, D), lambda c, i: (c, i, 0)),
            pl.BlockSpec((1, _RB, D), lambda c, i: (c, i, 0)),
            pl.BlockSpec((_RB, D), lambda c, i: (i, 0)),
            pl.BlockSpec((1, 1, D), lambda c, i: (c, 0, 0)),
            pl.BlockSpec((1, 1, _RB), lambda c, i: (i, 0, 0)),
        ],
        out_specs=pl.BlockSpec((1, B, D), lambda c, i: (c, 0, 0)),
        out_shape=jax.ShapeDtypeStruct((NC, B, D), jnp.float32),
    )(hp, seg, dinv, b, batch3)


def _tc_head_body(pooled_ref, w1_ref, b1_ref, w2_ref, b2_ref, out_ref):
    logits = []
    for c in range(NC):
        h = jnp.dot(pooled_ref[c], w1_ref[c], preferred_element_type=jnp.float32,
                 precision=lax.Precision.HIGHEST)
        h = jnp.maximum(h + b1_ref[c], 0.0)
        z = jnp.sum(h * w2_ref[c], axis=-1, keepdims=True)
        logits.append(z + b2_ref[c])
    z = jnp.concatenate(logits, axis=-1)
    m = jnp.max(z, axis=-1, keepdims=True)
    zs = z - m
    lse = jnp.log(jnp.sum(jnp.exp(zs), axis=-1, keepdims=True))
    out_ref[...] = zs - lse


def _tc_head(pooled, w1, b1, w2, b2):
    return pl.pallas_call(
        _tc_head_body,
        in_specs=[pl.BlockSpec(a.shape, functools.partial(lambda r: (0,) * r, a.ndim))
                  for a in (pooled, w1, b1, w2, b2)],
        out_specs=pl.BlockSpec((B, 2), lambda: (0, 0)),
        out_shape=jax.ShapeDtypeStruct((B, 2), jnp.float32),
    )(pooled, w1, b1, w2, b2)


def kernel(x, edge_index, batch, params):
    src, dst = edge_index[0], edge_index[1]
    x = jnp.pad(x, ((0, NP - N), (0, 0)))
    batch = jnp.pad(batch, (0, NP - N), constant_values=B)

    dst_g = dst.reshape(NC * NS, _DEG_CHUNKS, CHUNK)
    src_g = jnp.stack([src, src + NP]).reshape(NC, NS, _NGRP, _GRP_CHUNKS, CCHUNK)
    dst_g2 = dst.reshape(NS, _NGRP, _GRP_CHUNKS, CCHUNK)
    batch3 = batch.reshape(_NRB, 1, _RB)

    zeros128 = jnp.zeros((ROWS_PER_TILE, D), jnp.float32)
    ones128 = jnp.ones((CHUNK, D), jnp.float32)

    p0, p1 = params['c0'], params['c1']
    Ws = [jnp.stack([p0['convW'][l], p1['convW'][l]]) for l in range(3)]
    bs = [jnp.stack([p0['convb'][l], p1['convb'][l]]).reshape(NC, 1, D)
          for l in range(3)]
    w1 = jnp.stack([p0['lin1_W'], p1['lin1_W']])
    b1 = jnp.stack([p0['lin1_b'], p1['lin1_b']]).reshape(NC, 1, D)
    w2 = jnp.stack([p0['lin2_W'][:, 0], p1['lin2_W'][:, 0]]).reshape(NC, 1, D)
    b2 = jnp.stack([p0['lin2_b'], p1['lin2_b']]).reshape(NC, 1, 1)

    sc_degree, sc_conv = _sc_kernels()
    degp = sc_degree(dst_g, zeros128, ones128)
    dinv, hp = _tc_prep(degp, x, Ws[0])

    for l in range(3):
        seg = sc_conv(hp.reshape(NC * NP, D), src_g, dst_g2, zeros128)
        if l < 2:
            hp = _tc_layer(hp, seg, dinv, bs[l], Ws[l + 1])
        else:
            pooled = _tc_pool(hp, seg, dinv, bs[l], batch3)

    return _tc_head(pooled, w1, b1, w2, b2)


# speedup vs baseline: 1.2519x; 1.2519x over previous
import functools

import jax
import jax.numpy as jnp
from jax import lax
from jax.experimental import pallas as pl
from jax.experimental.pallas import tpu as pltpu
from jax.experimental.pallas import tpu_sc as plsc

N = 10000
NP = 10240
E = 320000
D = 128
B = 256
NC = 2
NS = 16
CHUNK = 80
CCHUNK = 40
NBUF = 5
ROWS_PER_TILE = NP // NS

_DEG_CHUNKS = E // (NC * NS) // CHUNK
_CONV_CHUNKS = E // NS // CCHUNK
_NGRP = 25
_GRP_CHUNKS = _CONV_CHUNKS // _NGRP
assert _GRP_CHUNKS % NBUF == 0
_NQUAD = _CONV_CHUNKS // NBUF
_QPG = _GRP_CHUNKS // NBUF


@functools.cache
def _sc_kernels():
    mesh = plsc.VectorSubcoreMesh(core_axis_name="c", subcore_axis_name="s",
                                  num_cores=NC, num_subcores=NS)

    @functools.partial(
        pl.kernel,
        mesh=mesh,
        out_type=jax.ShapeDtypeStruct((NC, NP, D), jnp.float32),
        scratch_types=[
            pltpu.VMEM((_DEG_CHUNKS, CHUNK), jnp.int32),
            pltpu.VMEM((CHUNK, D), jnp.float32),
            pltpu.VMEM_SHARED((NP, D), jnp.float32),
        ],
    )
    def sc_degree(dst_hbm, zeros_hbm, ones_hbm, out_hbm, idx_v, ones_v, deg_sh):
        cid = lax.axis_index("c")
        sid = lax.axis_index("s")
        wid = cid * NS + sid
        row0 = sid * ROWS_PER_TILE
        pltpu.sync_copy(zeros_hbm, deg_sh.at[pl.ds(row0, ROWS_PER_TILE)])
        pltpu.sync_copy(ones_hbm, ones_v)
        pltpu.sync_copy(dst_hbm.at[wid], idx_v)
        plsc.subcore_barrier()

        def body(j, carry):
            pltpu.sync_copy(ones_v, deg_sh.at[idx_v.at[j]], add=True)
            return carry

        lax.fori_loop(0, _DEG_CHUNKS, body, 0)
        plsc.subcore_barrier()
        pltpu.sync_copy(deg_sh.at[pl.ds(row0, ROWS_PER_TILE)],
                        out_hbm.at[cid, pl.ds(row0, ROWS_PER_TILE)])

    @functools.partial(
        pl.kernel,
        mesh=mesh,
        out_type=jax.ShapeDtypeStruct((NC, NP, D), jnp.float32),
        scratch_types=[
            pltpu.VMEM((2, _GRP_CHUNKS, CCHUNK), jnp.int32),
            pltpu.VMEM((2, _GRP_CHUNKS, CCHUNK), jnp.int32),
            pltpu.VMEM((NBUF, CCHUNK, D), jnp.float32),
            [pltpu.SemaphoreType.DMA for _ in range(NBUF)],
            [pltpu.SemaphoreType.DMA for _ in range(NBUF)],
            pltpu.SemaphoreType.DMA,
            pltpu.VMEM_SHARED((NP, D), jnp.float32),
        ],
    )
    def sc_conv(hp_hbm, srcg_hbm, dstg_hbm, zeros_hbm, out_hbm,
                src_v, dst_v, rows, gsems, ssems, isem, acc_sh):
        cid = lax.axis_index("c")
        sid = lax.axis_index("s")
        row0 = sid * ROWS_PER_TILE
        pltpu.sync_copy(zeros_hbm, acc_sh.at[pl.ds(row0, ROWS_PER_TILE)])
        pltpu.sync_copy(srcg_hbm.at[cid, sid, 0], src_v.at[0])
        pltpu.sync_copy(dstg_hbm.at[sid, 0], dst_v.at[0])
        plsc.subcore_barrier()

        def quad(q, carry):
            ra = (q % _QPG) * NBUF
            grp = q // _QPG
            gpar = grp % 2
            npar = (grp + 1) % 2

            @pl.when((ra == 0) & (grp > 0))
            def _wait_idx():
                pltpu.make_async_copy(srcg_hbm.at[cid, sid, grp],
                                      src_v.at[gpar], isem).wait()
                pltpu.make_async_copy(dstg_hbm.at[sid, grp],
                                      dst_v.at[gpar], isem).wait()

            @pl.when((ra == 0) & (grp < _NGRP - 1))
            def _prefetch_idx():
                pltpu.async_copy(srcg_hbm.at[cid, sid, grp + 1],
                                 src_v.at[npar], isem)
                pltpu.async_copy(dstg_hbm.at[sid, grp + 1],
                                 dst_v.at[npar], isem)

            for k in range(NBUF):
                @pl.when(q > 0)
                def _drain_prev(k=k):
                    pltpu.make_async_copy(
                        rows.at[k], acc_sh.at[dst_v.at[gpar, ra + k]],
                        ssems[k]).wait()
                pltpu.async_copy(hp_hbm.at[src_v.at[gpar, ra + k]],
                                 rows.at[k], gsems[k])
            for k in range(NBUF):
                pltpu.make_async_copy(hp_hbm.at[src_v.at[gpar, ra + k]],
                                      rows.at[k], gsems[k]).wait()
                pltpu.async_copy(rows.at[k], acc_sh.at[dst_v.at[gpar, ra + k]],
                                 ssems[k], add=True)
            return carry

        lax.fori_loop(0, _NQUAD, quad, 0)
        for k in range(NBUF):
            pltpu.make_async_copy(rows.at[k], acc_sh.at[dst_v.at[0, k]],
                                  ssems[k]).wait()
        plsc.subcore_barrier()
        pltpu.sync_copy(acc_sh.at[pl.ds(row0, ROWS_PER_TILE)],
                        out_hbm.at[cid, pl.ds(row0, ROWS_PER_TILE)])

    return sc_degree, sc_conv


_RB = 1024
_NRB = NP // _RB


def _tc_prep_body(deg_ref, x_ref, w_ref, dinv_ref, hp_ref):
    degf = deg_ref[0] + deg_ref[1]
    total = degf + 1.0
    dinv = lax.rsqrt(jnp.maximum(total, 1.0))
    dinv_ref[...] = dinv
    h = jnp.dot(x_ref[...], w_ref[0], preferred_element_type=jnp.float32,
                 precision=lax.Precision.HIGHEST)
    hp_ref[...] = (h * dinv)[None]


def _tc_prep(degp, x, w0):
    return pl.pallas_call(
        _tc_prep_body,
        grid=(NC, _NRB),
        in_specs=[
            pl.BlockSpec((NC, _RB, D), lambda c, i: (0, i, 0)),
            pl.BlockSpec((_RB, D), lambda c, i: (i, 0)),
            pl.BlockSpec((1, D, D), lambda c, i: (c, 0, 0)),
        ],
        out_specs=[
            pl.BlockSpec((_RB, D), lambda c, i: (i, 0)),
            pl.BlockSpec((1, _RB, D), lambda c, i: (c, i, 0)),
        ],
        out_shape=[
            jax.ShapeDtypeStruct((NP, D), jnp.float32),
            jax.ShapeDtypeStruct((NC, NP, D), jnp.float32),
        ],
    )(degp, x, w0)


def _combine(hp, seg, dinv, b):
    out = dinv * (hp + seg) + b
    nrm2 = jnp.sum(out * out, axis=-1, keepdims=True)
    return jnp.maximum(out * lax.rsqrt(jnp.maximum(nrm2, 1e-24)), 0.0)


def _tc_layer_body(hp_ref, seg_ref, dinv_ref, b_ref, w_ref, hpn_ref):
    xn = _combine(hp_ref[0], seg_ref[0], dinv_ref[...], b_ref[0])
    h = jnp.dot(xn, w_ref[0], preferred_element_type=jnp.float32,
                 precision=lax.Precision.HIGHEST)
    hpn_ref[...] = (h * dinv_ref[...])[None]


def _tc_layer(hp, seg, dinv, b, wn):
    return pl.pallas_call(
        _tc_layer_body,
        grid=(NC, _NRB),
        in_specs=[
            pl.BlockSpec((1, _RB, D), lambda c, i: (c, i, 0)),
            pl.BlockSpec((1, _RB, D), lambda c, i: (c, i, 0)),
            pl.BlockSpec((_RB, D), lambda c, i: (i, 0)),
            pl.BlockSpec((1, 1, D), lambda c, i: (c, 0, 0)),
            pl.BlockSpec((1, D, D), lambda c, i: (c, 0, 0)),
        ],
        out_specs=pl.BlockSpec((1, _RB, D), lambda c, i: (c, i, 0)),
        out_shape=jax.ShapeDtypeStruct((NC, NP, D), jnp.float32),
    )(hp, seg, dinv, b, wn)


def _tc_pool_body(hp_ref, seg_ref, dinv_ref, b_ref, batch_ref, pooled_ref):
    i = pl.program_id(1)
    xn = _combine(hp_ref[0], seg_ref[0], dinv_ref[...], b_ref[0])
    bvec = batch_ref[0, 0]
    gids = lax.broadcasted_iota(jnp.int32, (B, _RB), 0)
    oh_t = jnp.where(gids == bvec[None, :], 1.0, 0.0)
    contrib = jnp.dot(oh_t, xn, preferred_element_type=jnp.float32,
                 precision=lax.Precision.HIGHEST)

    @pl.when(i == 0)
    def _init():
        pooled_ref[...] = contrib[None]

    @pl.when(i != 0)
    def _acc():
        pooled_ref[...] = pooled_ref[...] + contrib[None]


def _tc_pool(hp, seg, dinv, b, batch3):
    return pl.pallas_call(
        _tc_pool_body,
        grid=(NC, _NRB),
        in_specs=[
            pl.BlockSpec((1, _RB, D), lambda c, i: (c, i, 0)),
            pl.BlockSpec((1, _RB, D), lambda c, i: (c, i, 0)),
            pl.BlockSpec((_RB, D), lambda c, i: (i, 0)),
            pl.BlockSpec((1, 1, D), lambda c, i: (c, 0, 0)),
            pl.BlockSpec((1, 1, _RB), lambda c, i: (i, 0, 0)),
        ],
        out_specs=pl.BlockSpec((1, B, D), lambda c, i: (c, 0, 0)),
        out_shape=jax.ShapeDtypeStruct((NC, B, D), jnp.float32),
    )(hp, seg, dinv, b, batch3)


def _tc_head_body(pooled_ref, w1_ref, b1_ref, w2_ref, b2_ref, out_ref):
    logits = []
    for c in range(NC):
        h = jnp.dot(pooled_ref[c], w1_ref[c], preferred_element_type=jnp.float32,
                 precision=lax.Precision.HIGHEST)
        h = jnp.maximum(h + b1_ref[c], 0.0)
        z = jnp.sum(h * w2_ref[c], axis=-1, keepdims=True)
        logits.append(z + b2_ref[c])
    z = jnp.concatenate(logits, axis=-1)
    m = jnp.max(z, axis=-1, keepdims=True)
    zs = z - m
    lse = jnp.log(jnp.sum(jnp.exp(zs), axis=-1, keepdims=True))
    out_ref[...] = zs - lse


def _tc_head(pooled, w1, b1, w2, b2):
    return pl.pallas_call(
        _tc_head_body,
        in_specs=[pl.BlockSpec(a.shape, functools.partial(lambda r: (0,) * r, a.ndim))
                  for a in (pooled, w1, b1, w2, b2)],
        out_specs=pl.BlockSpec((B, 2), lambda: (0, 0)),
        out_shape=jax.ShapeDtypeStruct((B, 2), jnp.float32),
    )(pooled, w1, b1, w2, b2)


def kernel(x, edge_index, batch, params):
    src, dst = edge_index[0], edge_index[1]
    x = jnp.pad(x, ((0, NP - N), (0, 0)))
    batch = jnp.pad(batch, (0, NP - N), constant_values=B)

    dst_g = dst.reshape(NC * NS, _DEG_CHUNKS, CHUNK)
    src_g = jnp.stack([src, src + NP]).reshape(NC, NS, _NGRP, _GRP_CHUNKS, CCHUNK)
    dst_g2 = dst.reshape(NS, _NGRP, _GRP_CHUNKS, CCHUNK)
    batch3 = batch.reshape(_NRB, 1, _RB)

    zeros128 = jnp.zeros((ROWS_PER_TILE, D), jnp.float32)
    ones128 = jnp.ones((CHUNK, D), jnp.float32)

    p0, p1 = params['c0'], params['c1']
    Ws = [jnp.stack([p0['convW'][l], p1['convW'][l]]) for l in range(3)]
    bs = [jnp.stack([p0['convb'][l], p1['convb'][l]]).reshape(NC, 1, D)
          for l in range(3)]
    w1 = jnp.stack([p0['lin1_W'], p1['lin1_W']])
    b1 = jnp.stack([p0['lin1_b'], p1['lin1_b']]).reshape(NC, 1, D)
    w2 = jnp.stack([p0['lin2_W'][:, 0], p1['lin2_W'][:, 0]]).reshape(NC, 1, D)
    b2 = jnp.stack([p0['lin2_b'], p1['lin2_b']]).reshape(NC, 1, 1)

    sc_degree, sc_conv = _sc_kernels()
    degp = sc_degree(dst_g, zeros128, ones128)
    dinv, hp = _tc_prep(degp, x, Ws[0])

    for l in range(3):
        seg = sc_conv(hp.reshape(NC * NP, D), src_g, dst_g2, zeros128)
        if l < 2:
            hp = _tc_layer(hp, seg, dinv, bs[l], Ws[l + 1])
        else:
            pooled = _tc_pool(hp, seg, dinv, bs[l], batch3)

    return _tc_head(pooled, w1, b1, w2, b2)


# speedup vs baseline: 1.2624x; 1.0084x over previous
import functools

import jax
import jax.numpy as jnp
from jax import lax
from jax.experimental import pallas as pl
from jax.experimental.pallas import tpu as pltpu
from jax.experimental.pallas import tpu_sc as plsc

N = 10000
NP = 10240
E = 320000
D = 128
B = 256
NC = 2
NS = 16
CHUNK = 80
DEGW = 128
CCHUNK = 40
NBUF = 5
ROWS_PER_TILE = NP // NS

_DEG_CHUNKS = E // (NC * NS) // CHUNK
_CONV_CHUNKS = E // NS // CCHUNK
_NGRP = 25
_GRP_CHUNKS = _CONV_CHUNKS // _NGRP
assert _GRP_CHUNKS % NBUF == 0
_NQUAD = _CONV_CHUNKS // NBUF
_QPG = _GRP_CHUNKS // NBUF


@functools.cache
def _sc_kernels():
    mesh = plsc.VectorSubcoreMesh(core_axis_name="c", subcore_axis_name="s",
                                  num_cores=NC, num_subcores=NS)

    @functools.partial(
        pl.kernel,
        mesh=mesh,
        out_type=jax.ShapeDtypeStruct((NC, NP, DEGW), jnp.float32),
        scratch_types=[
            pltpu.VMEM((_DEG_CHUNKS, CHUNK), jnp.int32),
            pltpu.VMEM((CHUNK, DEGW), jnp.float32),
            pltpu.VMEM_SHARED((NP, DEGW), jnp.float32),
        ],
    )
    def sc_degree(dst_hbm, zeros_hbm, ones_hbm, out_hbm, idx_v, ones_v, deg_sh):
        cid = lax.axis_index("c")
        sid = lax.axis_index("s")
        wid = cid * NS + sid
        row0 = sid * ROWS_PER_TILE
        pltpu.sync_copy(zeros_hbm, deg_sh.at[pl.ds(row0, ROWS_PER_TILE)])
        pltpu.sync_copy(ones_hbm, ones_v)
        pltpu.sync_copy(dst_hbm.at[wid], idx_v)
        plsc.subcore_barrier()

        def body(j, carry):
            pltpu.sync_copy(ones_v, deg_sh.at[idx_v.at[j]], add=True)
            return carry

        lax.fori_loop(0, _DEG_CHUNKS, body, 0)
        plsc.subcore_barrier()
        pltpu.sync_copy(deg_sh.at[pl.ds(row0, ROWS_PER_TILE)],
                        out_hbm.at[cid, pl.ds(row0, ROWS_PER_TILE)])

    @functools.partial(
        pl.kernel,
        mesh=mesh,
        out_type=jax.ShapeDtypeStruct((NC, NP, D), jnp.float32),
        scratch_types=[
            pltpu.VMEM((2, _GRP_CHUNKS, CCHUNK), jnp.int32),
            pltpu.VMEM((2, _GRP_CHUNKS, CCHUNK), jnp.int32),
            pltpu.VMEM((NBUF, CCHUNK, D), jnp.float32),
            [pltpu.SemaphoreType.DMA for _ in range(NBUF)],
            [pltpu.SemaphoreType.DMA for _ in range(NBUF)],
            pltpu.SemaphoreType.DMA,
            pltpu.VMEM_SHARED((NP, D), jnp.float32),
        ],
    )
    def sc_conv(hp_hbm, srcg_hbm, dstg_hbm, out_hbm,
                src_v, dst_v, rows, gsems, ssems, isem, acc_sh):
        cid = lax.axis_index("c")
        sid = lax.axis_index("s")
        row0 = sid * ROWS_PER_TILE
        pltpu.sync_copy(hp_hbm.at[pl.ds(cid * NP + row0, ROWS_PER_TILE)],
                        acc_sh.at[pl.ds(row0, ROWS_PER_TILE)])
        pltpu.sync_copy(srcg_hbm.at[cid, sid, 0], src_v.at[0])
        pltpu.sync_copy(dstg_hbm.at[sid, 0], dst_v.at[0])
        plsc.subcore_barrier()

        def quad(q, carry):
            ra = (q % _QPG) * NBUF
            grp = q // _QPG
            gpar = grp % 2
            npar = (grp + 1) % 2

            @pl.when((ra == 0) & (grp > 0))
            def _wait_idx():
                pltpu.make_async_copy(srcg_hbm.at[cid, sid, grp],
                                      src_v.at[gpar], isem).wait()
                pltpu.make_async_copy(dstg_hbm.at[sid, grp],
                                      dst_v.at[gpar], isem).wait()

            @pl.when((ra == 0) & (grp < _NGRP - 1))
            def _prefetch_idx():
                pltpu.async_copy(srcg_hbm.at[cid, sid, grp + 1],
                                 src_v.at[npar], isem)
                pltpu.async_copy(dstg_hbm.at[sid, grp + 1],
                                 dst_v.at[npar], isem)

            for k in range(NBUF):
                @pl.when(q > 0)
                def _drain_prev(k=k):
                    pltpu.make_async_copy(
                        rows.at[k], acc_sh.at[dst_v.at[gpar, ra + k]],
                        ssems[k]).wait()
                pltpu.async_copy(hp_hbm.at[src_v.at[gpar, ra + k]],
                                 rows.at[k], gsems[k])
            for k in range(NBUF):
                pltpu.make_async_copy(hp_hbm.at[src_v.at[gpar, ra + k]],
                                      rows.at[k], gsems[k]).wait()
                pltpu.async_copy(rows.at[k], acc_sh.at[dst_v.at[gpar, ra + k]],
                                 ssems[k], add=True)
            return carry

        lax.fori_loop(0, _NQUAD, quad, 0)
        for k in range(NBUF):
            pltpu.make_async_copy(rows.at[k], acc_sh.at[dst_v.at[0, k]],
                                  ssems[k]).wait()
        plsc.subcore_barrier()
        pltpu.sync_copy(acc_sh.at[pl.ds(row0, ROWS_PER_TILE)],
                        out_hbm.at[cid, pl.ds(row0, ROWS_PER_TILE)])

    return sc_degree, sc_conv


_RB = 1024
_NRB = NP // _RB


def _tc_prep_body(deg_ref, x_ref, w_ref, dinv_ref, hp_ref):
    degw = deg_ref[0] + deg_ref[1]
    degf = jnp.concatenate([degw] * (D // DEGW), axis=-1)
    total = degf + 1.0
    dinv = lax.rsqrt(jnp.maximum(total, 1.0))
    dinv_ref[...] = dinv
    h = jnp.dot(x_ref[...], w_ref[0], preferred_element_type=jnp.float32,
                 precision=lax.Precision.HIGHEST)
    hp_ref[...] = (h * dinv)[None]


def _tc_prep(degp, x, w0):
    return pl.pallas_call(
        _tc_prep_body,
        grid=(NC, _NRB),
        in_specs=[
            pl.BlockSpec((NC, _RB, DEGW), lambda c, i: (0, i, 0)),
            pl.BlockSpec((_RB, D), lambda c, i: (i, 0)),
            pl.BlockSpec((1, D, D), lambda c, i: (c, 0, 0)),
        ],
        out_specs=[
            pl.BlockSpec((_RB, D), lambda c, i: (i, 0)),
            pl.BlockSpec((1, _RB, D), lambda c, i: (c, i, 0)),
        ],
        out_shape=[
            jax.ShapeDtypeStruct((NP, D), jnp.float32),
            jax.ShapeDtypeStruct((NC, NP, D), jnp.float32),
        ],
    )(degp, x, w0)


def _combine(seg, dinv, b):
    out = dinv * seg + b
    nrm2 = jnp.sum(out * out, axis=-1, keepdims=True)
    return jnp.maximum(out * lax.rsqrt(jnp.maximum(nrm2, 1e-24)), 0.0)


def _tc_layer_body(seg_ref, dinv_ref, b_ref, w_ref, hpn_ref):
    xn = _combine(seg_ref[0], dinv_ref[...], b_ref[0])
    h = jnp.dot(xn, w_ref[0], preferred_element_type=jnp.float32,
                 precision=lax.Precision.HIGHEST)
    hpn_ref[...] = (h * dinv_ref[...])[None]


def _tc_layer(seg, dinv, b, wn):
    return pl.pallas_call(
        _tc_layer_body,
        grid=(NC, _NRB),
        in_specs=[
            pl.BlockSpec((1, _RB, D), lambda c, i: (c, i, 0)),
            pl.BlockSpec((_RB, D), lambda c, i: (i, 0)),
            pl.BlockSpec((1, 1, D), lambda c, i: (c, 0, 0)),
            pl.BlockSpec((1, D, D), lambda c, i: (c, 0, 0)),
        ],
        out_specs=pl.BlockSpec((1, _RB, D), lambda c, i: (c, i, 0)),
        out_shape=jax.ShapeDtypeStruct((NC, NP, D), jnp.float32),
    )(seg, dinv, b, wn)


def _tc_pool_body(seg_ref, dinv_ref, b_ref, batch_ref, pooled_ref):
    i = pl.program_id(1)
    xn = _combine(seg_ref[0], dinv_ref[...], b_ref[0])
    bvec = batch_ref[0, 0]
    gids = lax.broadcasted_iota(jnp.int32, (B, _RB), 0)
    oh_t = jnp.where(gids == bvec[None, :], 1.0, 0.0)
    contrib = jnp.dot(oh_t, xn, preferred_element_type=jnp.float32,
                 precision=lax.Precision.HIGHEST)

    @pl.when(i == 0)
    def _init():
        pooled_ref[...] = contrib[None]

    @pl.when(i != 0)
    def _acc():
        pooled_ref[...] = pooled_ref[...] + contrib[None]


def _tc_pool(seg, dinv, b, batch3):
    return pl.pallas_call(
        _tc_pool_body,
        grid=(NC, _NRB),
        in_specs=[
            pl.BlockSpec((1, _RB, D), lambda c, i: (c, i, 0)),
            pl.BlockSpec((_RB, D), lambda c, i: (i, 0)),
            pl.BlockSpec((1, 1, D), lambda c, i: (c, 0, 0)),
            pl.BlockSpec((1, 1, _RB), lambda c, i: (i, 0, 0)),
        ],
        out_specs=pl.BlockSpec((1, B, D), lambda c, i: (c, 0, 0)),
        out_shape=jax.ShapeDtypeStruct((NC, B, D), jnp.float32),
    )(seg, dinv, b, batch3)


def _tc_head_body(pooled_ref, w1_ref, b1_ref, w2_ref, b2_ref, out_ref):
    logits = []
    for c in range(NC):
        h = jnp.dot(pooled_ref[c], w1_ref[c], preferred_element_type=jnp.float32,
                 precision=lax.Precision.HIGHEST)
        h = jnp.maximum(h + b1_ref[c], 0.0)
        z = jnp.sum(h * w2_ref[c], axis=-1, keepdims=True)
        logits.append(z + b2_ref[c])
    z = jnp.concatenate(logits, axis=-1)
    m = jnp.max(z, axis=-1, keepdims=True)
    zs = z - m
    lse = jnp.log(jnp.sum(jnp.exp(zs), axis=-1, keepdims=True))
    out_ref[...] = zs - lse


def _tc_head(pooled, w1, b1, w2, b2):
    return pl.pallas_call(
        _tc_head_body,
        in_specs=[pl.BlockSpec(a.shape, functools.partial(lambda r: (0,) * r, a.ndim))
                  for a in (pooled, w1, b1, w2, b2)],
        out_specs=pl.BlockSpec((B, 2), lambda: (0, 0)),
        out_shape=jax.ShapeDtypeStruct((B, 2), jnp.float32),
    )(pooled, w1, b1, w2, b2)


def kernel(x, edge_index, batch, params):
    src, dst = edge_index[0], edge_index[1]
    x = jnp.pad(x, ((0, NP - N), (0, 0)))
    batch = jnp.pad(batch, (0, NP - N), constant_values=B)

    dst_g = dst.reshape(NC * NS, _DEG_CHUNKS, CHUNK)
    src_g = jnp.stack([src, src + NP]).reshape(NC, NS, _NGRP, _GRP_CHUNKS, CCHUNK)
    dst_g2 = dst.reshape(NS, _NGRP, _GRP_CHUNKS, CCHUNK)
    batch3 = batch.reshape(_NRB, 1, _RB)

    zeros_deg = jnp.zeros((ROWS_PER_TILE, DEGW), jnp.float32)
    ones_deg = jnp.ones((CHUNK, DEGW), jnp.float32)

    p0, p1 = params['c0'], params['c1']
    Ws = [jnp.stack([p0['convW'][l], p1['convW'][l]]) for l in range(3)]
    bs = [jnp.stack([p0['convb'][l], p1['convb'][l]]).reshape(NC, 1, D)
          for l in range(3)]
    w1 = jnp.stack([p0['lin1_W'], p1['lin1_W']])
    b1 = jnp.stack([p0['lin1_b'], p1['lin1_b']]).reshape(NC, 1, D)
    w2 = jnp.stack([p0['lin2_W'][:, 0], p1['lin2_W'][:, 0]]).reshape(NC, 1, D)
    b2 = jnp.stack([p0['lin2_b'], p1['lin2_b']]).reshape(NC, 1, 1)

    sc_degree, sc_conv = _sc_kernels()
    degp = sc_degree(dst_g, zeros_deg, ones_deg)
    dinv, hp = _tc_prep(degp, x, Ws[0])

    for l in range(3):
        seg = sc_conv(hp.reshape(NC * NP, D), src_g, dst_g2)
        if l < 2:
            hp = _tc_layer(seg, dinv, bs[l], Ws[l + 1])
        else:
            pooled = _tc_pool(seg, dinv, bs[l], batch3)

    return _tc_head(pooled, w1, b1, w2, b2)


# speedup vs baseline: 1.3978x; 1.1073x over previous
import functools

import jax
import jax.numpy as jnp
from jax import lax
from jax.experimental import pallas as pl
from jax.experimental.pallas import tpu as pltpu
from jax.experimental.pallas import tpu_sc as plsc

N = 10000
NP = 10240
E = 320000
D = 128
B = 256
NC = 2
NS = 16
CHUNK = 80
DEGW = 128
CCHUNK = 40
NBUF = 5
ROWS_PER_TILE = NP // NS

_DEG_CHUNKS = E // (NC * NS) // CHUNK
_CONV_CHUNKS = E // NS // CCHUNK
_NGRP = 25
_GRP_CHUNKS = _CONV_CHUNKS // _NGRP
assert _GRP_CHUNKS % NBUF == 0
_NQUAD = _CONV_CHUNKS // NBUF
_QPG = _GRP_CHUNKS // NBUF


@functools.cache
def _sc_kernels():
    mesh = plsc.VectorSubcoreMesh(core_axis_name="c", subcore_axis_name="s",
                                  num_cores=NC, num_subcores=NS)

    @functools.partial(
        pl.kernel,
        mesh=mesh,
        out_type=jax.ShapeDtypeStruct((NC, NP, DEGW), jnp.float32),
        scratch_types=[
            pltpu.VMEM((_DEG_CHUNKS, CHUNK), jnp.int32),
            pltpu.VMEM((CHUNK, DEGW), jnp.float32),
            pltpu.VMEM_SHARED((NP, DEGW), jnp.float32),
        ],
    )
    def sc_degree(dst_hbm, zeros_hbm, ones_hbm, out_hbm, idx_v, ones_v, deg_sh):
        cid = lax.axis_index("c")
        sid = lax.axis_index("s")
        wid = cid * NS + sid
        row0 = sid * ROWS_PER_TILE
        pltpu.sync_copy(zeros_hbm, deg_sh.at[pl.ds(row0, ROWS_PER_TILE)])
        pltpu.sync_copy(ones_hbm, ones_v)
        pltpu.sync_copy(dst_hbm.at[wid], idx_v)
        plsc.subcore_barrier()

        def body(j, carry):
            pltpu.sync_copy(ones_v, deg_sh.at[idx_v.at[j]], add=True)
            return carry

        lax.fori_loop(0, _DEG_CHUNKS, body, 0)
        plsc.subcore_barrier()
        pltpu.sync_copy(deg_sh.at[pl.ds(row0, ROWS_PER_TILE)],
                        out_hbm.at[cid, pl.ds(row0, ROWS_PER_TILE)])

    def make_conv(shared):
        nchunks = E // (NC * NS if shared else NS) // CCHUNK
        ngrp = _NGRP
        grp_chunks = nchunks // ngrp
        qpg = grp_chunks // NBUF
        assert grp_chunks % NBUF == 0
        nquad = nchunks // NBUF

        @functools.partial(
            pl.kernel,
            mesh=mesh,
            out_type=jax.ShapeDtypeStruct((NC, NP, D), jnp.float32),
            scratch_types=[
                pltpu.VMEM((2, grp_chunks, CCHUNK), jnp.int32),
                pltpu.VMEM((2, grp_chunks, CCHUNK), jnp.int32),
                pltpu.VMEM((NBUF, CCHUNK, D), jnp.float32),
                [pltpu.SemaphoreType.DMA for _ in range(NBUF)],
                [pltpu.SemaphoreType.DMA for _ in range(NBUF)],
                pltpu.SemaphoreType.DMA,
                pltpu.VMEM_SHARED((NP, D), jnp.float32),
            ],
        )
        def conv(tbl_hbm, srcg_hbm, dstg_hbm, init_hbm, out_hbm,
                 src_v, dst_v, rows, gsems, ssems, isem, acc_sh):
            cid = lax.axis_index("c")
            sid = lax.axis_index("s")
            row0 = sid * ROWS_PER_TILE
            if shared:
                pltpu.sync_copy(init_hbm, acc_sh.at[pl.ds(row0, ROWS_PER_TILE)])
            else:
                pltpu.sync_copy(tbl_hbm.at[pl.ds(cid * NP + row0, ROWS_PER_TILE)],
                                acc_sh.at[pl.ds(row0, ROWS_PER_TILE)])
            pltpu.sync_copy(srcg_hbm.at[cid, sid, 0], src_v.at[0])
            pltpu.sync_copy(dstg_hbm.at[cid, sid, 0], dst_v.at[0])
            plsc.subcore_barrier()

            def quad(q, carry):
                ra = (q % qpg) * NBUF
                grp = q // qpg
                gpar = grp % 2
                npar = (grp + 1) % 2

                @pl.when((ra == 0) & (grp > 0))
                def _wait_idx():
                    pltpu.make_async_copy(srcg_hbm.at[cid, sid, grp],
                                          src_v.at[gpar], isem).wait()
                    pltpu.make_async_copy(dstg_hbm.at[cid, sid, grp],
                                          dst_v.at[gpar], isem).wait()

                @pl.when((ra == 0) & (grp < ngrp - 1))
                def _prefetch_idx():
                    pltpu.async_copy(srcg_hbm.at[cid, sid, grp + 1],
                                     src_v.at[npar], isem)
                    pltpu.async_copy(dstg_hbm.at[cid, sid, grp + 1],
                                     dst_v.at[npar], isem)

                for k in range(NBUF):
                    @pl.when(q > 0)
                    def _drain_prev(k=k):
                        pltpu.make_async_copy(
                            rows.at[k], acc_sh.at[dst_v.at[gpar, ra + k]],
                            ssems[k]).wait()
                    pltpu.async_copy(tbl_hbm.at[src_v.at[gpar, ra + k]],
                                     rows.at[k], gsems[k])
                for k in range(NBUF):
                    pltpu.make_async_copy(tbl_hbm.at[src_v.at[gpar, ra + k]],
                                          rows.at[k], gsems[k]).wait()
                    pltpu.async_copy(rows.at[k], acc_sh.at[dst_v.at[gpar, ra + k]],
                                     ssems[k], add=True)
                return carry

            lax.fori_loop(0, nquad, quad, 0)
            for k in range(NBUF):
                pltpu.make_async_copy(rows.at[k], acc_sh.at[dst_v.at[0, k]],
                                      ssems[k]).wait()
            plsc.subcore_barrier()
            pltpu.sync_copy(acc_sh.at[pl.ds(row0, ROWS_PER_TILE)],
                            out_hbm.at[cid, pl.ds(row0, ROWS_PER_TILE)])

        return conv

    sc_conv = make_conv(shared=False)
    sc_conv0 = make_conv(shared=True)

    return sc_degree, sc_conv, sc_conv0


_RB = 1024
_NRB = NP // _RB


def _tc_prep0_body(deg_ref, x_ref, dinv_ref, xp_ref):
    degw = deg_ref[0] + deg_ref[1]
    degf = jnp.concatenate([degw] * (D // DEGW), axis=-1)
    total = degf + 1.0
    dinv = lax.rsqrt(jnp.maximum(total, 1.0))
    dinv_ref[...] = dinv
    xp_ref[...] = x_ref[...] * dinv


def _tc_prep0(degp, x):
    return pl.pallas_call(
        _tc_prep0_body,
        grid=(_NRB,),
        in_specs=[
            pl.BlockSpec((NC, _RB, DEGW), lambda i: (0, i, 0)),
            pl.BlockSpec((_RB, D), lambda i: (i, 0)),
        ],
        out_specs=[
            pl.BlockSpec((_RB, D), lambda i: (i, 0)),
            pl.BlockSpec((_RB, D), lambda i: (i, 0)),
        ],
        out_shape=[
            jax.ShapeDtypeStruct((NP, D), jnp.float32),
            jax.ShapeDtypeStruct((NP, D), jnp.float32),
        ],
    )(degp, x)


def _tc_first_body(y_ref, xp_ref, dinv_ref, w0_ref, b0_ref, w1_ref, hp_ref):
    dinv = dinv_ref[...]
    y = y_ref[0] + y_ref[1] + xp_ref[...]
    out = jnp.dot(y, w0_ref[0], preferred_element_type=jnp.float32,
                  precision=lax.Precision.HIGHEST)
    xn = _combine(out, dinv, b0_ref[0])
    h = jnp.dot(xn, w1_ref[0], preferred_element_type=jnp.float32,
                precision=lax.Precision.HIGHEST)
    hp_ref[...] = (h * dinv)[None]


def _tc_first(y, xp, dinv, w0, b0, w1):
    return pl.pallas_call(
        _tc_first_body,
        grid=(NC, _NRB),
        in_specs=[
            pl.BlockSpec((NC, _RB, D), lambda c, i: (0, i, 0)),
            pl.BlockSpec((_RB, D), lambda c, i: (i, 0)),
            pl.BlockSpec((_RB, D), lambda c, i: (i, 0)),
            pl.BlockSpec((1, D, D), lambda c, i: (c, 0, 0)),
            pl.BlockSpec((1, 1, D), lambda c, i: (c, 0, 0)),
            pl.BlockSpec((1, D, D), lambda c, i: (c, 0, 0)),
        ],
        out_specs=pl.BlockSpec((1, _RB, D), lambda c, i: (c, i, 0)),
        out_shape=jax.ShapeDtypeStruct((NC, NP, D), jnp.float32),
    )(y, xp, dinv, w0, b0, w1)


def _combine(seg, dinv, b):
    out = dinv * seg + b
    nrm2 = jnp.sum(out * out, axis=-1, keepdims=True)
    return jnp.maximum(out * lax.rsqrt(jnp.maximum(nrm2, 1e-24)), 0.0)


def _tc_layer_body(seg_ref, dinv_ref, b_ref, w_ref, hpn_ref):
    xn = _combine(seg_ref[0], dinv_ref[...], b_ref[0])
    h = jnp.dot(xn, w_ref[0], preferred_element_type=jnp.float32,
                 precision=lax.Precision.HIGHEST)
    hpn_ref[...] = (h * dinv_ref[...])[None]


def _tc_layer(seg, dinv, b, wn):
    return pl.pallas_call(
        _tc_layer_body,
        grid=(NC, _NRB),
        in_specs=[
            pl.BlockSpec((1, _RB, D), lambda c, i: (c, i, 0)),
            pl.BlockSpec((_RB, D), lambda c, i: (i, 0)),
            pl.BlockSpec((1, 1, D), lambda c, i: (c, 0, 0)),
            pl.BlockSpec((1, D, D), lambda c, i: (c, 0, 0)),
        ],
        out_specs=pl.BlockSpec((1, _RB, D), lambda c, i: (c, i, 0)),
        out_shape=jax.ShapeDtypeStruct((NC, NP, D), jnp.float32),
    )(seg, dinv, b, wn)


def _tc_pool_body(seg_ref, dinv_ref, b_ref, batch_ref, pooled_ref):
    i = pl.program_id(1)
    xn = _combine(seg_ref[0], dinv_ref[...], b_ref[0])
    bvec = batch_ref[0, 0]
    gids = lax.broadcasted_iota(jnp.int32, (B, _RB), 0)
    oh_t = jnp.where(gids == bvec[None, :], 1.0, 0.0)
    contrib = jnp.dot(oh_t, xn, preferred_element_type=jnp.float32,
                 precision=lax.Precision.HIGHEST)

    @pl.when(i == 0)
    def _init():
        pooled_ref[...] = contrib[None]

    @pl.when(i != 0)
    def _acc():
        pooled_ref[...] = pooled_ref[...] + contrib[None]


def _tc_pool(seg, dinv, b, batch3):
    return pl.pallas_call(
        _tc_pool_body,
        grid=(NC, _NRB),
        in_specs=[
            pl.BlockSpec((1, _RB, D), lambda c, i: (c, i, 0)),
            pl.BlockSpec((_RB, D), lambda c, i: (i, 0)),
            pl.BlockSpec((1, 1, D), lambda c, i: (c, 0, 0)),
            pl.BlockSpec((1, 1, _RB), lambda c, i: (i, 0, 0)),
        ],
        out_specs=pl.BlockSpec((1, B, D), lambda c, i: (c, 0, 0)),
        out_shape=jax.ShapeDtypeStruct((NC, B, D), jnp.float32),
    )(seg, dinv, b, batch3)


def _tc_head_body(pooled_ref, w1_ref, b1_ref, w2_ref, b2_ref, out_ref):
    logits = []
    for c in range(NC):
        h = jnp.dot(pooled_ref[c], w1_ref[c], preferred_element_type=jnp.float32,
                 precision=lax.Precision.HIGHEST)
        h = jnp.maximum(h + b1_ref[c], 0.0)
        z = jnp.sum(h * w2_ref[c], axis=-1, keepdims=True)
        logits.append(z + b2_ref[c])
    z = jnp.concatenate(logits, axis=-1)
    m = jnp.max(z, axis=-1, keepdims=True)
    zs = z - m
    lse = jnp.log(jnp.sum(jnp.exp(zs), axis=-1, keepdims=True))
    out_ref[...] = zs - lse


def _tc_head(pooled, w1, b1, w2, b2):
    return pl.pallas_call(
        _tc_head_body,
        in_specs=[pl.BlockSpec(a.shape, functools.partial(lambda r: (0,) * r, a.ndim))
                  for a in (pooled, w1, b1, w2, b2)],
        out_specs=pl.BlockSpec((B, 2), lambda: (0, 0)),
        out_shape=jax.ShapeDtypeStruct((B, 2), jnp.float32),
    )(pooled, w1, b1, w2, b2)


def kernel(x, edge_index, batch, params):
    src, dst = edge_index[0], edge_index[1]
    x = jnp.pad(x, ((0, NP - N), (0, 0)))
    batch = jnp.pad(batch, (0, NP - N), constant_values=B)

    dst_g = dst.reshape(NC * NS, _DEG_CHUNKS, CHUNK)
    src_g = jnp.stack([src, src + NP]).reshape(NC, NS, _NGRP, _GRP_CHUNKS, CCHUNK)
    dst_g_b = jnp.broadcast_to(
        dst.reshape(1, NS, _NGRP, _GRP_CHUNKS, CCHUNK),
        (NC, NS, _NGRP, _GRP_CHUNKS, CCHUNK))
    _sgc = _GRP_CHUNKS // 2
    srcg_sh = src.reshape(NC, NS, _NGRP, _sgc, CCHUNK)
    dstg_sh = dst.reshape(NC, NS, _NGRP, _sgc, CCHUNK)
    batch3 = batch.reshape(_NRB, 1, _RB)

    zeros_deg = jnp.zeros((ROWS_PER_TILE, DEGW), jnp.float32)
    ones_deg = jnp.ones((CHUNK, DEGW), jnp.float32)

    p0, p1 = params['c0'], params['c1']
    Ws = [jnp.stack([p0['convW'][l], p1['convW'][l]]) for l in range(3)]
    bs = [jnp.stack([p0['convb'][l], p1['convb'][l]]).reshape(NC, 1, D)
          for l in range(3)]
    w1 = jnp.stack([p0['lin1_W'], p1['lin1_W']])
    b1 = jnp.stack([p0['lin1_b'], p1['lin1_b']]).reshape(NC, 1, D)
    w2 = jnp.stack([p0['lin2_W'][:, 0], p1['lin2_W'][:, 0]]).reshape(NC, 1, D)
    b2 = jnp.stack([p0['lin2_b'], p1['lin2_b']]).reshape(NC, 1, 1)

    sc_degree, sc_conv, sc_conv0 = _sc_kernels()
    degp = sc_degree(dst_g, zeros_deg, ones_deg)
    dinv, xp = _tc_prep0(degp, x)

    zinit = jnp.zeros((ROWS_PER_TILE, D), jnp.float32)
    y = sc_conv0(xp, srcg_sh, dstg_sh, zinit)
    hp = _tc_first(y, xp, dinv, Ws[0], bs[0], Ws[1])

    for l in (1, 2):
        seg = sc_conv(hp.reshape(NC * NP, D), src_g, dst_g_b, zinit)
        if l < 2:
            hp = _tc_layer(seg, dinv, bs[l], Ws[l + 1])
        else:
            pooled = _tc_pool(seg, dinv, bs[l], batch3)

    return _tc_head(pooled, w1, b1, w2, b2)
